# baseline (speedup 1.0000x reference)
import functools

import jax
import jax.numpy as jnp
from jax import lax
from jax.experimental import pallas as pl
from jax.experimental.pallas import tpu as pltpu
from jax.experimental.pallas import tpu_sc as plsc

EMBED_DIM = 64


@functools.cache
def _make_lookup(B, D, n_frozen):
    info = plsc.get_sparse_core_info()
    NC, NS, L = info.num_cores, info.num_subcores, info.num_lanes
    NW = NC * NS
    assert B % (8 * NW) == 0 and D % L == 0
    b_per_w = B // NW
    G = 128
    assert b_per_w % G == 0
    n_chunks = b_per_w // G
    mesh = plsc.VectorSubcoreMesh(core_axis_name="c", subcore_axis_name="s")

    @functools.partial(
        pl.kernel,
        mesh=mesh,
        out_type=jax.ShapeDtypeStruct((B, D), jnp.float32),
        compiler_params=pltpu.CompilerParams(use_tc_tiling_on_sc=False),
        scratch_types=[
            pltpu.VMEM((b_per_w,), jnp.int32),
            pltpu.VMEM((G,), jnp.int32),
            pltpu.VMEM((G,), jnp.int32),
            pltpu.VMEM((G, D), jnp.float32),
            pltpu.VMEM((G, D), jnp.float32),
            pltpu.SemaphoreType.DMA,
            pltpu.SemaphoreType.DMA,
        ],
    )
    def lookup(ids_hbm, frozen_hbm, weight_hbm, out_hbm,
               idx_v, idxf, idxt, rows_f, rows_t, semf, semt):
        wid = lax.axis_index("s") * NC + lax.axis_index("c")
        base = wid * b_per_w
        pltpu.sync_copy(ids_hbm.at[pl.ds(base, b_per_w)], idx_v)

        def chunk_body(c, carry):
            for kk in range(G // L):
                v = idx_v[pl.ds(c * G + kk * L, L)]
                m = v < n_frozen
                idxf[pl.ds(kk * L, L)] = jnp.where(m, v, 0)
                idxt[pl.ds(kk * L, L)] = jnp.where(m, 0, v - n_frozen)
            cf = pltpu.async_copy(frozen_hbm.at[idxf], rows_f, semf)
            ct = pltpu.async_copy(weight_hbm.at[idxt], rows_t, semt)
            cf.wait()
            ct.wait()

            def blend_grp(kk, rcarry):
                v = idx_v[pl.ds(c * G + kk * L, L)]
                for lane in range(L):
                    r = kk * L + lane

                    @pl.when(v[lane] >= n_frozen)
                    def _():
                        for d in range(D // L):
                            rows_f[r, pl.ds(d * L, L)] = rows_t[r, pl.ds(d * L, L)]

                return rcarry

            lax.fori_loop(0, G // L, blend_grp, 0)
            pltpu.sync_copy(rows_f, out_hbm.at[pl.ds(base + c * G, G)])
            return carry

        lax.fori_loop(0, n_chunks, chunk_body, 0)

    return lookup


def kernel(input, frozen_weight, weight):
    ids = input.reshape(-1).astype(jnp.int32)
    lookup = _make_lookup(ids.shape[0], EMBED_DIM, frozen_weight.shape[0])
    out = lookup(ids, frozen_weight, weight)
    return out.reshape(input.shape + (EMBED_DIM,))


# speedup vs baseline: 7.7321x; 7.7321x over previous
import functools

import jax
import jax.numpy as jnp
from jax import lax
from jax.experimental import pallas as pl
from jax.experimental.pallas import tpu as pltpu
from jax.experimental.pallas import tpu_sc as plsc

EMBED_DIM = 64
G = 128
LOG2G = 7
NB = 4
AHEAD = 2


@functools.cache
def _make_lookup(B, D, n_frozen):
    info = plsc.get_sparse_core_info()
    NC, NS, L = info.num_cores, info.num_subcores, info.num_lanes
    NW = NC * NS
    assert B % (8 * NW) == 0 and D % L == 0
    b_per_w = B // NW
    assert b_per_w % G == 0
    n_max = b_per_w // G + 2
    assert n_max % NB == 0
    n_grp = n_max // NB
    CP = n_max * G + L
    mesh = plsc.VectorSubcoreMesh(core_axis_name="c", subcore_axis_name="s")

    @functools.partial(
        pl.kernel,
        mesh=mesh,
        out_type=jax.ShapeDtypeStruct((B, D), jnp.float32),
        compiler_params=pltpu.CompilerParams(use_tc_tiling_on_sc=False,
                                             needs_layout_passes=False),
        scratch_types=[
            pltpu.VMEM((b_per_w,), jnp.int32),
            pltpu.VMEM((CP,), jnp.int32),
            pltpu.VMEM((NB, G), jnp.int32),
            pltpu.VMEM((NB, G), jnp.int32),
            pltpu.VMEM((NB, G, D), jnp.float32),
            pltpu.SemaphoreType.DMA((NB,)),
            pltpu.SemaphoreType.DMA((NB,)),
        ],
    )
    def lookup(ids_hbm, frozen_hbm, weight_hbm, out_hbm,
               idx_v, cpos, idx_sc, pos_sc, rows, semg, sems):
        wid = lax.axis_index("s") * NC + lax.axis_index("c")
        base = wid * b_per_w
        pltpu.sync_copy(ids_hbm.at[pl.ds(base, b_per_w)], idx_v)
        lanes = lax.iota(jnp.int32, L)

        def compact(offset, want_frozen):
            def body(i, carry):
                cnt, anyv = carry
                v = idx_v[pl.ds(i * L, L)]
                m = (v < n_frozen) if want_frozen else (v >= n_frozen)
                local = i * L + lanes
                _, sv = plsc.sort_key_val(v, local,
                                          descending=not want_frozen)
                cpos[pl.ds(offset + cnt, L)] = sv
                pc = plsc.all_reduce_population_count(m)[0]
                anyv = jnp.where(pc > 0, sv[0], anyv)
                return cnt + pc, anyv
            return lax.fori_loop(0, b_per_w // L, body,
                                 (jnp.int32(0), jnp.int32(0)))

        def pad(start, end, fill):
            fill_v = jnp.broadcast_to(fill, (L,))
            for k in range(G // L):
                pos = start + k * L

                @pl.when(pos < end)
                def _():
                    cpos[pl.ds(pos, L)] = fill_v

        nf, lf = compact(jnp.int32(0), True)
        nfc = (nf + G - 1) >> LOG2G
        nf_pad = nfc * G
        pad(nf, nf_pad, lf)
        nt, lt = compact(nf_pad, False)
        ntc = (nt + G - 1) >> LOG2G
        tc = nfc + ntc
        pad(nf_pad + nt, tc * G, lt)

        def prep(jn, b2):
            jn_eff = jnp.minimum(jn, tc - 1)
            is_fn = jn_eff < nfc
            s = jn_eff * G
            sub = jnp.where(is_fn, 0, n_frozen)
            for k in range(G // L):
                local = cpos[pl.ds(s + k * L, L)]
                iv = plsc.load_gather(idx_v, [local])
                idx_sc[b2, pl.ds(k * L, L)] = iv - sub
                pos_sc[b2, pl.ds(k * L, L)] = local + base

            @pl.when(is_fn)
            def _():
                pltpu.make_async_copy(
                    frozen_hbm.at[idx_sc.at[b2]], rows.at[b2], semg.at[b2]
                ).start()

            @pl.when(jnp.logical_not(is_fn))
            def _():
                pltpu.make_async_copy(
                    weight_hbm.at[idx_sc.at[b2]], rows.at[b2], semg.at[b2]
                ).start()

        for b in range(AHEAD):
            prep(jnp.int32(b), b)

        def grp(g, carry):
            for b in range(NB):
                j = g * NB + b
                pltpu.make_async_copy(
                    frozen_hbm.at[idx_sc.at[b]], rows.at[b], semg.at[b]
                ).wait()
                pltpu.make_async_copy(
                    rows.at[b], out_hbm.at[pos_sc.at[b]], sems.at[b]
                ).start()
                b2 = (b + AHEAD) % NB

                @pl.when(j + AHEAD < n_max)
                def _():
                    @pl.when(j >= NB - AHEAD)
                    def _():
                        pltpu.make_async_copy(
                            rows.at[b2], out_hbm.at[pos_sc.at[b2]],
                            sems.at[b2]
                        ).wait()

                    prep(j + AHEAD, b2)

            return carry

        lax.fori_loop(0, n_grp, grp, 0)

        for b in range(NB):
            pltpu.make_async_copy(
                rows.at[b], out_hbm.at[pos_sc.at[b]], sems.at[b]
            ).wait()

    return lookup


def kernel(input, frozen_weight, weight):
    ids = input.reshape(-1).astype(jnp.int32)
    lookup = _make_lookup(ids.shape[0], EMBED_DIM, frozen_weight.shape[0])
    out = lookup(ids, frozen_weight, weight)
    return out.reshape(input.shape + (EMBED_DIM,))


# speedup vs baseline: 8.0548x; 1.0417x over previous
import functools

import jax
import jax.numpy as jnp
from jax import lax
from jax.experimental import pallas as pl
from jax.experimental.pallas import tpu as pltpu
from jax.experimental.pallas import tpu_sc as plsc

EMBED_DIM = 64
G = 128
LOG2G = 7
NB = 13
AHEAD = 8


@functools.cache
def _make_lookup(B, D, n_frozen):
    info = plsc.get_sparse_core_info()
    NC, NS, L = info.num_cores, info.num_subcores, info.num_lanes
    NW = NC * NS
    assert B % (8 * NW) == 0 and D % L == 0
    b_per_w = B // NW
    assert b_per_w % G == 0
    n_max = b_per_w // G + 2
    assert n_max % NB == 0
    n_grp = n_max // NB
    CP = n_max * G + L
    mesh = plsc.VectorSubcoreMesh(core_axis_name="c", subcore_axis_name="s")

    @functools.partial(
        pl.kernel,
        mesh=mesh,
        out_type=jax.ShapeDtypeStruct((B, D), jnp.float32),
        compiler_params=pltpu.CompilerParams(use_tc_tiling_on_sc=False,
                                             needs_layout_passes=False),
        scratch_types=[
            pltpu.VMEM((b_per_w,), jnp.int32),
            pltpu.VMEM((CP,), jnp.int32),
            pltpu.VMEM((NB, G), jnp.int32),
            pltpu.VMEM((NB, G), jnp.int32),
            pltpu.VMEM((NB, G, D), jnp.float32),
            pltpu.SemaphoreType.DMA((NB,)),
            pltpu.SemaphoreType.DMA((NB,)),
        ],
    )
    def lookup(ids_hbm, frozen_hbm, weight_hbm, out_hbm,
               idx_v, cpos, idx_sc, pos_sc, rows, semg, sems):
        wid = lax.axis_index("s") * NC + lax.axis_index("c")
        base = wid * b_per_w
        pltpu.sync_copy(ids_hbm.at[pl.ds(base, b_per_w)], idx_v)
        lanes = lax.iota(jnp.int32, L)

        def compact(offset, want_frozen):
            def body(i, carry):
                cnt, anyv = carry
                v = idx_v[pl.ds(i * L, L)]
                m = (v < n_frozen) if want_frozen else (v >= n_frozen)
                local = i * L + lanes
                _, sv = plsc.sort_key_val(v, local,
                                          descending=not want_frozen)
                cpos[pl.ds(offset + cnt, L)] = sv
                pc = plsc.all_reduce_population_count(m)[0]
                anyv = jnp.where(pc > 0, sv[0], anyv)
                return cnt + pc, anyv
            return lax.fori_loop(0, b_per_w // L, body,
                                 (jnp.int32(0), jnp.int32(0)))

        def pad(start, end, fill):
            fill_v = jnp.broadcast_to(fill, (L,))
            for k in range(G // L):
                pos = start + k * L

                @pl.when(pos < end)
                def _():
                    cpos[pl.ds(pos, L)] = fill_v

        nf, lf = compact(jnp.int32(0), True)
        nfc = (nf + G - 1) >> LOG2G
        nf_pad = nfc * G
        pad(nf, nf_pad, lf)
        nt, lt = compact(nf_pad, False)
        ntc = (nt + G - 1) >> LOG2G
        tc = nfc + ntc
        pad(nf_pad + nt, tc * G, lt)

        def prep(jn, b2):
            jn_eff = jnp.minimum(jn, tc - 1)
            is_fn = jn_eff < nfc
            s = jn_eff * G
            sub = jnp.where(is_fn, 0, n_frozen)
            for k in range(G // L):
                local = cpos[pl.ds(s + k * L, L)]
                iv = plsc.load_gather(idx_v, [local])
                idx_sc[b2, pl.ds(k * L, L)] = iv - sub
                pos_sc[b2, pl.ds(k * L, L)] = local + base

            @pl.when(is_fn)
            def _():
                pltpu.make_async_copy(
                    frozen_hbm.at[idx_sc.at[b2]], rows.at[b2], semg.at[b2]
                ).start()

            @pl.when(jnp.logical_not(is_fn))
            def _():
                pltpu.make_async_copy(
                    weight_hbm.at[idx_sc.at[b2]], rows.at[b2], semg.at[b2]
                ).start()

        for b in range(AHEAD):
            prep(jnp.int32(b), b)

        def grp(g, carry):
            for b in range(NB):
                j = g * NB + b
                pltpu.make_async_copy(
                    frozen_hbm.at[idx_sc.at[b]], rows.at[b], semg.at[b]
                ).wait()
                pltpu.make_async_copy(
                    rows.at[b], out_hbm.at[pos_sc.at[b]], sems.at[b]
                ).start()
                b2 = (b + AHEAD) % NB

                @pl.when(j + AHEAD < n_max)
                def _():
                    @pl.when(j >= NB - AHEAD)
                    def _():
                        pltpu.make_async_copy(
                            rows.at[b2], out_hbm.at[pos_sc.at[b2]],
                            sems.at[b2]
                        ).wait()

                    prep(j + AHEAD, b2)

            return carry

        lax.fori_loop(0, n_grp, grp, 0)

        for b in range(NB):
            pltpu.make_async_copy(
                rows.at[b], out_hbm.at[pos_sc.at[b]], sems.at[b]
            ).wait()

    return lookup


def kernel(input, frozen_weight, weight):
    ids = input.reshape(-1).astype(jnp.int32)
    lookup = _make_lookup(ids.shape[0], EMBED_DIM, frozen_weight.shape[0])
    out = lookup(ids, frozen_weight, weight)
    return out.reshape(input.shape + (EMBED_DIM,))
